# baseline (speedup 1.0000x reference)
import functools
import jax
import jax.numpy as jnp
from jax import lax
from jax.experimental import pallas as pl
from jax.experimental.pallas import tpu as pltpu
from jax.experimental.pallas import tpu_sc as plsc

B = 16384
H = 20
D = 16
L = 16
NC = 2
NS = 16
NW = NC * NS
PER_W = B // NW
S = 128
NCHUNK = PER_W // S
ST = S * H
SD = S * D


NU = 1000000
CT_TILES = (NU + 127) // 128
CT_PER_W = (CT_TILES + NW - 1) // NW
TW = 2048
NFLAT = CT_TILES * TW


def _pack_body(ut, itbl, uF, iF, in_t, out_t, sem):
    wid = lax.axis_index("s") * NC + lax.axis_index("c")
    lanes = lax.iota(jnp.int32, L)
    lo = jnp.minimum(wid * CT_PER_W, CT_TILES)
    hi = jnp.minimum(lo + CT_PER_W, CT_TILES)

    def table(src, dst):
        def ct_body(ct, carry):
            pltpu.sync_copy(src.at[:, pl.ds(pl.multiple_of(ct * 128, 128), 128)],
                            in_t)

            def rl_body(rl, c2):
                out_t[pl.ds(rl * D, D)] = plsc.load_gather(
                    in_t, [lanes, jnp.full((L,), rl, jnp.int32)])
                return c2

            lax.fori_loop(0, 128, rl_body, 0)
            pltpu.sync_copy(out_t, dst.at[pl.ds(ct * TW, TW)])
            return carry

        lax.fori_loop(lo, hi, ct_body, 0)

    table(ut, uF)
    table(itbl, iF)


@functools.lru_cache(maxsize=1)
def _pack_call():
  return pl.kernel(
    _pack_body,
    out_type=(jax.ShapeDtypeStruct((NFLAT,), jnp.float32),
              jax.ShapeDtypeStruct((NFLAT,), jnp.float32)),
    mesh=plsc.VectorSubcoreMesh(core_axis_name="c", subcore_axis_name="s",
                                num_cores=NC, num_subcores=NS),
    scratch_types=[
        pltpu.VMEM((D, 128), jnp.float32),
        pltpu.VMEM((TW,), jnp.float32),
        pltpu.SemaphoreType.DMA,
    ],
    compiler_params=pltpu.CompilerParams(needs_layout_passes=False,
                                         use_tc_tiling_on_sc=True),
  )


def _score_body(user, item, it_in, uF, iF, u_bias, i_bias, t_embed, out,
                uidx, iidx, ueidx, ieidx, tidx, u_rows, i_rows, t_rows,
                ub, ib, prod_t, out_v, sem):
    wid = lax.axis_index("s") * NC + lax.axis_index("c")
    lanes = lax.iota(jnp.int32, L)

    for j in range(NCHUNK):
        base = wid * PER_W + j * S
        pltpu.sync_copy(user.at[pl.ds(base, S)], uidx)
        pltpu.sync_copy(item.at[pl.ds(base, S)], iidx)
        pltpu.sync_copy(it_in.at[pl.ds(base * H, ST)], tidx)

        def eidx(e, carry):
            ecol = jnp.full((L,), e, jnp.int32)
            ueidx[pl.ds(e * D, D)] = plsc.load_gather(uidx, [ecol]) * D + lanes
            ieidx[pl.ds(e * D, D)] = plsc.load_gather(iidx, [ecol]) * D + lanes
            return carry

        lax.fori_loop(0, S, eidx, 0)

        cps = [
            pltpu.async_copy(t_embed.at[tidx], t_rows, sem),
            pltpu.async_copy(uF.at[ueidx], u_rows, sem),
            pltpu.async_copy(iF.at[ieidx], i_rows, sem),
            pltpu.async_copy(u_bias.at[uidx], ub, sem),
            pltpu.async_copy(i_bias.at[iidx], ib, sem),
        ]
        for cp in cps:
            cp.wait()

        def example(e, carry):
            tb = e * H
            acc = t_rows[tb, :]
            for h in range(1, H):
                acc = acc + t_rows[tb + h, :]
            itv = i_rows[pl.ds(e * D, D)] + acc * (1.0 / H)
            prod_t[pl.ds(e * D, D)] = u_rows[pl.ds(e * D, D)] * itv
            return carry

        lax.fori_loop(0, S, example, 0)

        def group(g, carry):
            acc = plsc.load_gather(prod_t, [lanes * D + g * (L * D)])
            for d in range(1, D):
                acc = acc + plsc.load_gather(prod_t, [lanes * D + (g * (L * D) + d)])
            out_v[pl.ds(g * L, L)] = acc + ub[pl.ds(g * L, L)] + ib[pl.ds(g * L, L)]
            return carry

        lax.fori_loop(0, S // L, group, 0)
        pltpu.sync_copy(out_v, out.at[pl.ds(base, S)])


@functools.lru_cache(maxsize=1)
def _score_call():
  return pl.kernel(
    _score_body,
    out_type=jax.ShapeDtypeStruct((B,), jnp.float32),
    mesh=plsc.VectorSubcoreMesh(core_axis_name="c", subcore_axis_name="s",
                                num_cores=NC, num_subcores=NS),
    scratch_types=[
        pltpu.VMEM((S,), jnp.int32),
        pltpu.VMEM((S,), jnp.int32),
        pltpu.VMEM((SD,), jnp.int32),
        pltpu.VMEM((SD,), jnp.int32),
        pltpu.VMEM((ST,), jnp.int32),
        pltpu.VMEM((SD,), jnp.float32),
        pltpu.VMEM((SD,), jnp.float32),
        pltpu.VMEM((ST, D), jnp.float32),
        pltpu.VMEM((S,), jnp.float32),
        pltpu.VMEM((S,), jnp.float32),
        pltpu.VMEM((SD,), jnp.float32),
        pltpu.VMEM((S,), jnp.float32),
        pltpu.SemaphoreType.DMA,
    ],
    compiler_params=pltpu.CompilerParams(needs_layout_passes=False,
                                         use_tc_tiling_on_sc=False),
  )


@jax.jit
def kernel(user, item, it_in, it_off, u_bias, i_bias, u_embed, i_embed, t_embed):
    del it_off
    uF, iF = _pack_call()(u_embed.T, i_embed.T)
    return _score_call()(user, item, it_in, uF, iF,
                         u_bias.reshape(-1), i_bias.reshape(-1), t_embed)


# speedup vs baseline: 1.4423x; 1.4423x over previous
import functools
import jax
import jax.numpy as jnp
from jax import lax
from jax.experimental import pallas as pl
from jax.experimental.pallas import tpu as pltpu
from jax.experimental.pallas import tpu_sc as plsc

B = 16384
H = 20
D = 16
L = 16
NC = 2
NS = 16
NW = NC * NS
PER_W = B // NW
S = 128
NCHUNK = PER_W // S
ST = S * H
SD = S * D


NU = 1000000
CT_TILES = (NU + 127) // 128
CT_PER_W = (CT_TILES + NW - 1) // NW
TW = 2048
NFLAT = CT_TILES * TW


NBUF = 4
CT_QUADS = (CT_PER_W + NBUF - 1) // NBUF


def _pack_body(ut, itbl, uF, iF,
               in0, in1, in2, in3, ot0, ot1, ot2, ot3,
               si0, si1, si2, si3, so0, so1, so2, so3):
    wid = lax.axis_index("s") * NC + lax.axis_index("c")
    lanes = lax.iota(jnp.int32, L)
    lo = jnp.minimum(wid * CT_PER_W, CT_TILES)
    hi = jnp.minimum(lo + CT_PER_W, CT_TILES)
    ins = [in0, in1, in2, in3]
    ots = [ot0, ot1, ot2, ot3]
    sis = [si0, si1, si2, si3]
    sos = [so0, so1, so2, so3]

    def table(src, dst):
        def fetch(ct, b):
            @pl.when(ct < hi)
            def _():
                pltpu.async_copy(
                    src.at[:, pl.ds(pl.multiple_of(ct * 128, 128), 128)],
                    ins[b], sis[b])

        for b in range(NBUF):
            fetch(lo + b, b)

        def quad(q, carry):
            ct0 = lo + q * NBUF
            for b in range(NBUF):
                ct = ct0 + b

                @pl.when(ct < hi)
                def _():
                    pltpu.make_async_copy(src.at[:, pl.ds(0, 128)],
                                          ins[b], sis[b]).wait()

                    @pl.when(q > 0)
                    def _():
                        pltpu.make_async_copy(ots[b], dst.at[pl.ds(0, TW)],
                                              sos[b]).wait()

                    def rl16(rb, c2):
                        for rr in range(L):
                            rl = rb * L + rr
                            ots[b][pl.ds(rl * D, D)] = plsc.load_gather(
                                ins[b], [lanes, jnp.full((L,), rl, jnp.int32)])
                        return c2

                    lax.fori_loop(0, 128 // L, rl16, 0)
                    pltpu.async_copy(ots[b], dst.at[pl.ds(ct * TW, TW)], sos[b])
                    fetch(ct + NBUF, b)
            return carry

        lax.fori_loop(0, CT_QUADS, quad, 0)
        for b in range(NBUF):
            @pl.when(lo + b < hi)
            def _():
                pltpu.make_async_copy(ots[b], dst.at[pl.ds(0, TW)],
                                      sos[b]).wait()

    table(ut, uF)
    table(itbl, iF)


@functools.lru_cache(maxsize=1)
def _pack_call():
  return pl.kernel(
    _pack_body,
    out_type=(jax.ShapeDtypeStruct((NFLAT,), jnp.float32),
              jax.ShapeDtypeStruct((NFLAT,), jnp.float32)),
    mesh=plsc.VectorSubcoreMesh(core_axis_name="c", subcore_axis_name="s",
                                num_cores=NC, num_subcores=NS),
    scratch_types=(
        [pltpu.VMEM((D, 128), jnp.float32)] * NBUF
        + [pltpu.VMEM((TW,), jnp.float32)] * NBUF
        + [pltpu.SemaphoreType.DMA] * (2 * NBUF)
    ),
    compiler_params=pltpu.CompilerParams(needs_layout_passes=False,
                                         use_tc_tiling_on_sc=True),
  )


def _score_body(user, item, it_in, uF, iF, u_bias, i_bias, t_embed, out,
                uidx, iidx, ueidx, ieidx, tidx, u_rows, i_rows, t_rows,
                ub, ib, prod_t, out_v, sem):
    wid = lax.axis_index("s") * NC + lax.axis_index("c")
    lanes = lax.iota(jnp.int32, L)

    for j in range(NCHUNK):
        base = wid * PER_W + j * S
        pltpu.sync_copy(user.at[pl.ds(base, S)], uidx)
        pltpu.sync_copy(item.at[pl.ds(base, S)], iidx)
        pltpu.sync_copy(it_in.at[pl.ds(base * H, ST)], tidx)

        def eidx(e, carry):
            ecol = jnp.full((L,), e, jnp.int32)
            ueidx[pl.ds(e * D, D)] = plsc.load_gather(uidx, [ecol]) * D + lanes
            ieidx[pl.ds(e * D, D)] = plsc.load_gather(iidx, [ecol]) * D + lanes
            return carry

        lax.fori_loop(0, S, eidx, 0)

        cps = [
            pltpu.async_copy(t_embed.at[tidx], t_rows, sem),
            pltpu.async_copy(uF.at[ueidx], u_rows, sem),
            pltpu.async_copy(iF.at[ieidx], i_rows, sem),
            pltpu.async_copy(u_bias.at[uidx], ub, sem),
            pltpu.async_copy(i_bias.at[iidx], ib, sem),
        ]
        for cp in cps:
            cp.wait()

        def example(e, carry):
            tb = e * H
            acc = t_rows[tb, :]
            for h in range(1, H):
                acc = acc + t_rows[tb + h, :]
            itv = i_rows[pl.ds(e * D, D)] + acc * (1.0 / H)
            prod_t[pl.ds(e * D, D)] = u_rows[pl.ds(e * D, D)] * itv
            return carry

        lax.fori_loop(0, S, example, 0)

        def group(g, carry):
            acc = plsc.load_gather(prod_t, [lanes * D + g * (L * D)])
            for d in range(1, D):
                acc = acc + plsc.load_gather(prod_t, [lanes * D + (g * (L * D) + d)])
            out_v[pl.ds(g * L, L)] = acc + ub[pl.ds(g * L, L)] + ib[pl.ds(g * L, L)]
            return carry

        lax.fori_loop(0, S // L, group, 0)
        pltpu.sync_copy(out_v, out.at[pl.ds(base, S)])


@functools.lru_cache(maxsize=1)
def _score_call():
  return pl.kernel(
    _score_body,
    out_type=jax.ShapeDtypeStruct((B,), jnp.float32),
    mesh=plsc.VectorSubcoreMesh(core_axis_name="c", subcore_axis_name="s",
                                num_cores=NC, num_subcores=NS),
    scratch_types=[
        pltpu.VMEM((S,), jnp.int32),
        pltpu.VMEM((S,), jnp.int32),
        pltpu.VMEM((SD,), jnp.int32),
        pltpu.VMEM((SD,), jnp.int32),
        pltpu.VMEM((ST,), jnp.int32),
        pltpu.VMEM((SD,), jnp.float32),
        pltpu.VMEM((SD,), jnp.float32),
        pltpu.VMEM((ST, D), jnp.float32),
        pltpu.VMEM((S,), jnp.float32),
        pltpu.VMEM((S,), jnp.float32),
        pltpu.VMEM((SD,), jnp.float32),
        pltpu.VMEM((S,), jnp.float32),
        pltpu.SemaphoreType.DMA,
    ],
    compiler_params=pltpu.CompilerParams(needs_layout_passes=False,
                                         use_tc_tiling_on_sc=False),
  )


@jax.jit
def kernel(user, item, it_in, it_off, u_bias, i_bias, u_embed, i_embed, t_embed):
    del it_off
    uF, iF = _pack_call()(u_embed.T, i_embed.T)
    return _score_call()(user, item, it_in, uF, iF,
                         u_bias.reshape(-1), i_bias.reshape(-1), t_embed)


# speedup vs baseline: 3.7504x; 2.6003x over previous
import functools
import jax
import jax.numpy as jnp
from jax import lax
from jax.experimental import pallas as pl
from jax.experimental.pallas import tpu as pltpu
from jax.experimental.pallas import tpu_sc as plsc

B = 16384
H = 20
D = 16
L = 16
NC = 2
NS = 16
NW = NC * NS
PER_W = B // NW
S = 128
NCHUNK = PER_W // S
ST = S * H
SD = S * D


NU = 1000000
CT_TILES = (NU + 127) // 128
CT_PER_W = (CT_TILES + NW - 1) // NW
TW = 2048
NFLAT = CT_TILES * TW


NBUF = 4
CT_QUADS = (CT_PER_W + NBUF - 1) // NBUF


def _pack_body(ut, itbl, uF, iF,
               in0, in1, in2, in3, ot0, ot1, ot2, ot3,
               si0, si1, si2, si3, so0, so1, so2, so3):
    wid = lax.axis_index("s") * NC + lax.axis_index("c")
    lanes = lax.iota(jnp.int32, L)
    lo = jnp.minimum(wid * CT_PER_W, CT_TILES)
    hi = jnp.minimum(lo + CT_PER_W, CT_TILES)
    ins = [in0, in1, in2, in3]
    ots = [ot0, ot1, ot2, ot3]
    sis = [si0, si1, si2, si3]
    sos = [so0, so1, so2, so3]

    def table(src, dst):
        def fetch(ct, b):
            @pl.when(ct < hi)
            def _():
                pltpu.async_copy(
                    src.at[:, pl.ds(pl.multiple_of(ct * 128, 128), 128)],
                    ins[b], sis[b])

        for b in range(NBUF):
            fetch(lo + b, b)

        def quad(q, carry):
            ct0 = lo + q * NBUF
            for b in range(NBUF):
                ct = ct0 + b

                @pl.when(ct < hi)
                def _():
                    pltpu.make_async_copy(src.at[:, pl.ds(0, 128)],
                                          ins[b], sis[b]).wait()

                    @pl.when(q > 0)
                    def _():
                        pltpu.make_async_copy(ots[b], dst.at[pl.ds(0, TW)],
                                              sos[b]).wait()

                    lanesD = lanes * D

                    def rl16(rb, c2):
                        colbase = lanesD + rb * (L * D)
                        for d in range(D):
                            val = ins[b][d, pl.ds(rb * L, L)]
                            plsc.store_scatter(ots[b], [colbase + d], val)
                        return c2

                    lax.fori_loop(0, 128 // L, rl16, 0)
                    pltpu.async_copy(ots[b], dst.at[pl.ds(ct * TW, TW)], sos[b])
                    fetch(ct + NBUF, b)
            return carry

        lax.fori_loop(0, CT_QUADS, quad, 0)
        for b in range(NBUF):
            @pl.when(lo + b < hi)
            def _():
                pltpu.make_async_copy(ots[b], dst.at[pl.ds(0, TW)],
                                      sos[b]).wait()

    table(ut, uF)
    table(itbl, iF)


@functools.lru_cache(maxsize=1)
def _pack_call():
  return pl.kernel(
    _pack_body,
    out_type=(jax.ShapeDtypeStruct((NFLAT,), jnp.float32),
              jax.ShapeDtypeStruct((NFLAT,), jnp.float32)),
    mesh=plsc.VectorSubcoreMesh(core_axis_name="c", subcore_axis_name="s",
                                num_cores=NC, num_subcores=NS),
    scratch_types=(
        [pltpu.VMEM((D, 128), jnp.float32)] * NBUF
        + [pltpu.VMEM((TW,), jnp.float32)] * NBUF
        + [pltpu.SemaphoreType.DMA] * (2 * NBUF)
    ),
    compiler_params=pltpu.CompilerParams(needs_layout_passes=False,
                                         use_tc_tiling_on_sc=True),
  )


def _score_body(user, item, it_in, uF, iF, u_bias, i_bias, t_embed, out,
                uidx, iidx, ueidx, ieidx, tidx, u_rows, i_rows, t_rows,
                ub, ib, prod_t, out_v, sem):
    wid = lax.axis_index("s") * NC + lax.axis_index("c")
    lanes = lax.iota(jnp.int32, L)

    for j in range(NCHUNK):
        base = wid * PER_W + j * S
        pltpu.sync_copy(user.at[pl.ds(base, S)], uidx)
        pltpu.sync_copy(item.at[pl.ds(base, S)], iidx)
        pltpu.sync_copy(it_in.at[pl.ds(base * H, ST)], tidx)

        def eidx(e, carry):
            ecol = jnp.full((L,), e, jnp.int32)
            ueidx[pl.ds(e * D, D)] = plsc.load_gather(uidx, [ecol]) * D + lanes
            ieidx[pl.ds(e * D, D)] = plsc.load_gather(iidx, [ecol]) * D + lanes
            return carry

        lax.fori_loop(0, S, eidx, 0)

        cps = [
            pltpu.async_copy(t_embed.at[tidx], t_rows, sem),
            pltpu.async_copy(uF.at[ueidx], u_rows, sem),
            pltpu.async_copy(iF.at[ieidx], i_rows, sem),
            pltpu.async_copy(u_bias.at[uidx], ub, sem),
            pltpu.async_copy(i_bias.at[iidx], ib, sem),
        ]
        for cp in cps:
            cp.wait()

        def example(e, carry):
            tb = e * H
            acc = t_rows[tb, :]
            for h in range(1, H):
                acc = acc + t_rows[tb + h, :]
            itv = i_rows[pl.ds(e * D, D)] + acc * (1.0 / H)
            prod_t[pl.ds(e * D, D)] = u_rows[pl.ds(e * D, D)] * itv
            return carry

        lax.fori_loop(0, S, example, 0)

        def group(g, carry):
            acc = plsc.load_gather(prod_t, [lanes * D + g * (L * D)])
            for d in range(1, D):
                acc = acc + plsc.load_gather(prod_t, [lanes * D + (g * (L * D) + d)])
            out_v[pl.ds(g * L, L)] = acc + ub[pl.ds(g * L, L)] + ib[pl.ds(g * L, L)]
            return carry

        lax.fori_loop(0, S // L, group, 0)
        pltpu.sync_copy(out_v, out.at[pl.ds(base, S)])


@functools.lru_cache(maxsize=1)
def _score_call():
  return pl.kernel(
    _score_body,
    out_type=jax.ShapeDtypeStruct((B,), jnp.float32),
    mesh=plsc.VectorSubcoreMesh(core_axis_name="c", subcore_axis_name="s",
                                num_cores=NC, num_subcores=NS),
    scratch_types=[
        pltpu.VMEM((S,), jnp.int32),
        pltpu.VMEM((S,), jnp.int32),
        pltpu.VMEM((SD,), jnp.int32),
        pltpu.VMEM((SD,), jnp.int32),
        pltpu.VMEM((ST,), jnp.int32),
        pltpu.VMEM((SD,), jnp.float32),
        pltpu.VMEM((SD,), jnp.float32),
        pltpu.VMEM((ST, D), jnp.float32),
        pltpu.VMEM((S,), jnp.float32),
        pltpu.VMEM((S,), jnp.float32),
        pltpu.VMEM((SD,), jnp.float32),
        pltpu.VMEM((S,), jnp.float32),
        pltpu.SemaphoreType.DMA,
    ],
    compiler_params=pltpu.CompilerParams(needs_layout_passes=False,
                                         use_tc_tiling_on_sc=False),
  )


@jax.jit
def kernel(user, item, it_in, it_off, u_bias, i_bias, u_embed, i_embed, t_embed):
    del it_off
    uF, iF = _pack_call()(u_embed.T, i_embed.T)
    return _score_call()(user, item, it_in, uF, iF,
                         u_bias.reshape(-1), i_bias.reshape(-1), t_embed)
